# baseline (speedup 1.0000x reference)
import functools

import jax
import jax.numpy as jnp
from jax import lax
from jax.experimental import pallas as pl
from jax.experimental.pallas import tpu as pltpu
from jax.experimental.pallas import tpu_sc as plsc

_NC, _NS = 2, 16
_NW = _NC * _NS
_D = 64
_M = 8
_CHUNK = 160


def _pack_codebook(codebook):
    cb16 = codebook.astype(jnp.bfloat16)
    g0 = jnp.stack([cb16[:, 0:16], cb16[:, 16:32]], axis=-1)
    g1 = jnp.stack([cb16[:, 32:48], cb16[:, 48:64]], axis=-1)
    return jnp.stack([g0.reshape(-1, 32), g1.reshape(-1, 32)], axis=1)


@functools.partial(jax.jit, static_argnums=(3, 4))
def _run(x_flat, codes_t_flat, cb_packed, total_words, vocab):
    wpt = total_words // _NW
    chunk = min(_CHUNK, wpt)
    n_chunks = wpt // chunk
    assert n_chunks % 2 == 0 or n_chunks == 1
    mesh = plsc.VectorSubcoreMesh(core_axis_name="c", subcore_axis_name="s")

    @functools.partial(
        pl.kernel,
        out_type=jax.ShapeDtypeStruct((total_words, _D), jnp.float32),
        mesh=mesh,
        compiler_params=pltpu.CompilerParams(
            needs_layout_passes=False, use_tc_tiling_on_sc=False),
        scratch_types=[
            pltpu.VMEM((wpt,), jnp.int32),
            pltpu.VMEM((chunk * _M,), jnp.int32),
            pltpu.VMEM((chunk * _M,), jnp.int32),
            pltpu.VMEM((chunk * _M,), jnp.int32),
            pltpu.VMEM((chunk * _M,), jnp.int32),
            pltpu.VMEM((chunk * _M, 2, 32), jnp.bfloat16),
            pltpu.VMEM((chunk * _M, 2, 32), jnp.bfloat16),
            pltpu.VMEM((chunk, _D), jnp.float32),
            pltpu.VMEM((chunk, _D), jnp.float32),
            pltpu.SemaphoreType.DMA,
            pltpu.SemaphoreType.DMA,
            pltpu.SemaphoreType.DMA,
            pltpu.SemaphoreType.DMA,
            pltpu.SemaphoreType.DMA,
            pltpu.SemaphoreType.DMA,
        ],
    )
    def run(x_hbm, codes_hbm, cbp_hbm, out_hbm, x_all,
            idx0, idx1, codes0, codes1, rows0, rows1, outv0, outv1,
            semc0, semc1, semr0, semr1, semo0, semo1):
        idx_b = (idx0, idx1)
        codes_b = (codes0, codes1)
        rows_b = (rows0, rows1)
        out_b = (outv0, outv1)
        semc = (semc0, semc1)
        semr = (semr0, semr1)
        semo = (semo0, semo1)

        wid = lax.axis_index("s") * _NC + lax.axis_index("c")
        base = wid * wpt
        pltpu.sync_copy(x_hbm.at[pl.ds(base, wpt)], x_all)

        lane = lax.iota(jnp.int32, 16)
        sub_w = lax.shift_right_logical(lane, 3)
        sub_m = lax.bitwise_and(lane, 7) * vocab

        def codes_gather(b):
            return pltpu.make_async_copy(
                codes_hbm.at[idx_b[b]], codes_b[b], semc[b])

        def rows_gather(b):
            return pltpu.make_async_copy(
                cbp_hbm.at[codes_b[b]], rows_b[b], semr[b])

        def out_copy(k, b):
            return pltpu.make_async_copy(
                out_b[b], out_hbm.at[pl.ds(base + k * chunk, chunk)], semo[b])

        def s1(k, b):
            def idx_body(j, c2):
                xg = plsc.load_gather(x_all, [k * chunk + 2 * j + sub_w])
                idx_b[b][pl.ds(16 * j, 16)] = xg + sub_m
                return c2

            lax.fori_loop(0, chunk * _M // 16, idx_body, 0)
            codes_gather(b).start()

        def s2(b):
            codes_gather(b).wait()
            rows_gather(b).start()

        def s3(k, b):
            rows_gather(b).wait()

            @pl.when(k >= 2)
            def _():
                out_copy(k - 2, b).wait()

            rows = rows_b[b]
            outv = out_b[b]

            def word_body(w, c2):
                r = w * _M
                for g in range(2):
                    s0 = rows[r + 0, g, :] + rows[r + 1, g, :]
                    t0 = rows[r + 2, g, :] + rows[r + 3, g, :]
                    s1_ = rows[r + 4, g, :] + rows[r + 5, g, :]
                    t1 = rows[r + 6, g, :] + rows[r + 7, g, :]
                    acc = (s0 + t0) + (s1_ + t1)
                    a, b_ = plsc.unpack(acc, format=plsc.PackFormat.INTERLEAVED)
                    outv[w, pl.ds(32 * g, 16)] = a
                    outv[w, pl.ds(32 * g + 16, 16)] = b_
                return c2

            lax.fori_loop(0, chunk, word_body, 0)
            out_copy(k, b).start()

        def pair_body(ii, carry):
            for u in (0, 1):
                i = 2 * ii + u

                @pl.when(i < n_chunks)
                def _():
                    s1(i, u)

                @pl.when(jnp.logical_and(i >= 1, i <= n_chunks))
                def _():
                    s2(1 - u)

                @pl.when(jnp.logical_and(i >= 2, i <= n_chunks + 1))
                def _():
                    s3(i - 2, u)
            return carry

        lax.fori_loop(0, (n_chunks + 2) // 2, pair_body, 0)
        out_copy(n_chunks - 2, 0).wait()
        out_copy(n_chunks - 1, 1).wait()

    return run(x_flat, codes_t_flat, cb_packed)


def kernel(x, codes, codebook):
    bsz, seq = x.shape
    total = bsz * seq
    vocab = codes.shape[0]
    x_flat = x.reshape(total).astype(jnp.int32)
    codes_t_flat = codes.T.reshape(-1)
    cb_packed = _pack_codebook(codebook)
    out = _run(x_flat, codes_t_flat, cb_packed, total, vocab)
    return out.reshape(bsz, seq, _D)


# speedup vs baseline: 1.1953x; 1.1953x over previous
import functools

import jax
import jax.numpy as jnp
from jax import lax
from jax.experimental import pallas as pl
from jax.experimental.pallas import tpu as pltpu
from jax.experimental.pallas import tpu_sc as plsc

_NC, _NS = 2, 16
_NW = _NC * _NS
_D = 64
_M = 8
_ROWS = 2


def _pack_codebook(codebook):
    cb16 = codebook.astype(jnp.bfloat16)
    g0 = jnp.stack([cb16[:, 0:16], cb16[:, 16:32]], axis=-1)
    g1 = jnp.stack([cb16[:, 32:48], cb16[:, 48:64]], axis=-1)
    return jnp.stack([g0.reshape(-1, 32), g1.reshape(-1, 32)], axis=1)


@functools.partial(jax.jit, static_argnums=(3, 4))
def _run(x_flat, codes, cb_packed, bsz, seq):
    total_words = bsz * seq
    wpt = total_words // _NW
    chunk = _ROWS * seq
    n_chunks = wpt // chunk
    rpt = bsz // _NW
    assert n_chunks * chunk == wpt and n_chunks % 2 == 0
    mesh = plsc.VectorSubcoreMesh(core_axis_name="c", subcore_axis_name="s")

    @functools.partial(
        pl.kernel,
        out_type=jax.ShapeDtypeStruct((bsz, seq, _D), jnp.float32),
        mesh=mesh,
        compiler_params=pltpu.CompilerParams(
            needs_layout_passes=False, use_tc_tiling_on_sc=False),
        scratch_types=[
            pltpu.VMEM((wpt + 16,), jnp.int32),
            pltpu.VMEM((chunk + 8, _M), jnp.int32),
            pltpu.VMEM((chunk + 8, _M), jnp.int32),
            pltpu.VMEM((chunk * _M,), jnp.int32),
            pltpu.VMEM((chunk * _M,), jnp.int32),
            pltpu.VMEM((chunk * _M, 2, 32), jnp.bfloat16),
            pltpu.VMEM((chunk * _M, 2, 32), jnp.bfloat16),
            pltpu.VMEM((_ROWS, seq, _D), jnp.float32),
            pltpu.VMEM((_ROWS, seq, _D), jnp.float32),
            pltpu.SemaphoreType.DMA,
            pltpu.SemaphoreType.DMA,
            pltpu.SemaphoreType.DMA,
            pltpu.SemaphoreType.DMA,
            pltpu.SemaphoreType.DMA,
            pltpu.SemaphoreType.DMA,
        ],
    )
    def run(x_hbm, codes_hbm, cbp_hbm, out_hbm, x_all,
            codes0, codes1, flat0, flat1, rows0, rows1, outv0, outv1,
            semc0, semc1, semr0, semr1, semo0, semo1):
        codes_b = (codes0, codes1)
        flat_b = (flat0, flat1)
        rows_b = (rows0, rows1)
        out_b = (outv0, outv1)
        semc = (semc0, semc1)
        semr = (semr0, semr1)
        semo = (semo0, semo1)

        wid = lax.axis_index("s") * _NC + lax.axis_index("c")
        base = wid * wpt
        row_base = wid * rpt
        pltpu.sync_copy(x_hbm.at[pl.ds(base, wpt)], x_all.at[pl.ds(0, wpt)])
        x_all[pl.ds(wpt, 16)] = jnp.zeros((16,), jnp.int32)

        lane = lax.iota(jnp.int32, 16)
        sub_w = lax.shift_right_logical(lane, 3)
        sub_m = lax.bitwise_and(lane, 7)

        def codes_gather(k, b):
            off = k * chunk
            aligned = pl.multiple_of(lax.bitwise_and(off, ~7), 8)
            return pltpu.make_async_copy(
                codes_hbm.at[x_all.at[pl.ds(aligned, chunk + 8)]],
                codes_b[b], semc[b])

        def rows_gather(b):
            return pltpu.make_async_copy(
                cbp_hbm.at[flat_b[b]], rows_b[b], semr[b])

        def out_copy(k, b):
            return pltpu.make_async_copy(
                out_b[b],
                out_hbm.at[pl.ds(row_base + k * _ROWS, _ROWS)], semo[b])

        def s1(i, b):
            codes_gather(i, b).start()

        def s2(k, b):
            codes_gather(k, b).wait()
            d0 = lax.bitwise_and(k * chunk, 7)

            def flat_body(j, c2):
                v = plsc.load_gather(codes_b[b], [d0 + 2 * j + sub_w, sub_m])
                flat_b[b][pl.ds(16 * j, 16)] = v
                return c2

            lax.fori_loop(0, chunk * _M // 16, flat_body, 0)
            rows_gather(b).start()

        def s3(k, b):
            rows_gather(b).wait()

            @pl.when(k >= 2)
            def _():
                out_copy(k - 2, b).wait()

            rows = rows_b[b]
            outv = out_b[b]

            for r in range(_ROWS):
                def word_body(l, c2, r=r):
                    p = (r * seq + l) * _M
                    for g in range(2):
                        s0 = rows[p + 0, g, :] + rows[p + 1, g, :]
                        t0 = rows[p + 2, g, :] + rows[p + 3, g, :]
                        s1_ = rows[p + 4, g, :] + rows[p + 5, g, :]
                        t1 = rows[p + 6, g, :] + rows[p + 7, g, :]
                        acc = (s0 + t0) + (s1_ + t1)
                        a, b_ = plsc.unpack(
                            acc, format=plsc.PackFormat.INTERLEAVED)
                        outv[r, l, pl.ds(32 * g, 16)] = a
                        outv[r, l, pl.ds(32 * g + 16, 16)] = b_
                    return c2

                lax.fori_loop(0, seq, word_body, 0)
            out_copy(k, b).start()

        def pair_body(ii, carry):
            for u in (0, 1):
                i = 2 * ii + u

                @pl.when(i < n_chunks)
                def _():
                    s1(i, u)

                @pl.when(jnp.logical_and(i >= 1, i <= n_chunks))
                def _():
                    s2(i - 1, 1 - u)

                @pl.when(jnp.logical_and(i >= 2, i <= n_chunks + 1))
                def _():
                    s3(i - 2, u)
            return carry

        lax.fori_loop(0, (n_chunks + 2) // 2, pair_body, 0)
        out_copy(n_chunks - 2, 0).wait()
        out_copy(n_chunks - 1, 1).wait()

    return run(x_flat, codes, cb_packed)


def kernel(x, codes, codebook):
    bsz, seq = x.shape
    x_flat = x.reshape(bsz * seq).astype(jnp.int32)
    cb_packed = _pack_codebook(codebook)
    return _run(x_flat, codes, cb_packed, bsz, seq)


# speedup vs baseline: 1.3244x; 1.1080x over previous
import functools

import jax
import jax.numpy as jnp
from jax import lax
from jax.experimental import pallas as pl
from jax.experimental.pallas import tpu as pltpu
from jax.experimental.pallas import tpu_sc as plsc

_NC, _NS = 2, 16
_NW = _NC * _NS
_D = 64
_M = 8
_CHUNK = 128


def _pack_codebook(codebook):
    cb16 = codebook.astype(jnp.bfloat16)
    g0 = jnp.stack([cb16[:, 0:16], cb16[:, 16:32]], axis=-1)
    g1 = jnp.stack([cb16[:, 32:48], cb16[:, 48:64]], axis=-1)
    return jnp.stack([g0.reshape(-1, 32), g1.reshape(-1, 32)], axis=1)


@functools.partial(jax.jit, static_argnums=(3,))
def _run(x_flat, codes, cb_packed, total_words):
    wpt = total_words // _NW
    chunk = min(_CHUNK, wpt)
    n_chunks = wpt // chunk
    assert n_chunks % 2 == 0 and chunk % 8 == 0
    vocab = cb_packed.shape[0]
    mesh = plsc.VectorSubcoreMesh(core_axis_name="c", subcore_axis_name="s")

    @functools.partial(
        pl.kernel,
        out_type=jax.ShapeDtypeStruct((total_words, _D), jnp.float32),
        mesh=mesh,
        compiler_params=pltpu.CompilerParams(
            needs_layout_passes=False, use_tc_tiling_on_sc=False),
        scratch_types=[
            pltpu.VMEM_SHARED((vocab, 2, 32), jnp.bfloat16),
            pltpu.VMEM((wpt,), jnp.int32),
            pltpu.VMEM((chunk, _M), jnp.int32),
            pltpu.VMEM((chunk, _M), jnp.int32),
            pltpu.VMEM((chunk * _M,), jnp.int32),
            pltpu.VMEM((chunk * _M,), jnp.int32),
            pltpu.VMEM((chunk * _M, 2, 32), jnp.bfloat16),
            pltpu.VMEM((chunk * _M, 2, 32), jnp.bfloat16),
            pltpu.VMEM((chunk, _D), jnp.float32),
            pltpu.VMEM((chunk, _D), jnp.float32),
            pltpu.SemaphoreType.DMA,
            pltpu.SemaphoreType.DMA,
            pltpu.SemaphoreType.DMA,
            pltpu.SemaphoreType.DMA,
            pltpu.SemaphoreType.DMA,
            pltpu.SemaphoreType.DMA,
        ],
    )
    def run(x_hbm, codes_hbm, cbp_hbm, out_hbm, cb_sh, x_all,
            codes0, codes1, flat0, flat1, rows0, rows1, outv0, outv1,
            semc0, semc1, semr0, semr1, semo0, semo1):
        codes_b = (codes0, codes1)
        flat_b = (flat0, flat1)
        rows_b = (rows0, rows1)
        out_b = (outv0, outv1)
        semc = (semc0, semc1)
        semr = (semr0, semr1)
        semo = (semo0, semo1)

        sid = lax.axis_index("s")
        wid = sid * _NC + lax.axis_index("c")
        base = wid * wpt

        @pl.when(sid == 0)
        def _():
            pltpu.sync_copy(cbp_hbm, cb_sh)

        pltpu.sync_copy(x_hbm.at[pl.ds(base, wpt)], x_all)
        plsc.subcore_barrier()

        lane = lax.iota(jnp.int32, 16)
        sub_w = lax.shift_right_logical(lane, 3)
        sub_m = lax.bitwise_and(lane, 7)

        def codes_gather(k, b):
            return pltpu.make_async_copy(
                codes_hbm.at[x_all.at[pl.ds(k * chunk, chunk)]],
                codes_b[b], semc[b])

        def rows_gather(b):
            return pltpu.make_async_copy(
                cb_sh.at[flat_b[b]], rows_b[b], semr[b])

        def out_copy(k, b):
            return pltpu.make_async_copy(
                out_b[b], out_hbm.at[pl.ds(base + k * chunk, chunk)], semo[b])

        def s1(i, b):
            codes_gather(i, b).start()

        def s2(k, b):
            codes_gather(k, b).wait()

            def flat_body(j, c2):
                v = plsc.load_gather(codes_b[b], [2 * j + sub_w, sub_m])
                flat_b[b][pl.ds(16 * j, 16)] = v
                return c2

            lax.fori_loop(0, chunk * _M // 16, flat_body, 0)
            rows_gather(b).start()

        def s3(k, b):
            rows_gather(b).wait()

            @pl.when(k >= 2)
            def _():
                out_copy(k - 2, b).wait()

            rows = rows_b[b]
            outv = out_b[b]

            def word_body(w, c2):
                p = w * _M
                for g in range(2):
                    s0 = rows[p + 0, g, :] + rows[p + 1, g, :]
                    t0 = rows[p + 2, g, :] + rows[p + 3, g, :]
                    s1_ = rows[p + 4, g, :] + rows[p + 5, g, :]
                    t1 = rows[p + 6, g, :] + rows[p + 7, g, :]
                    acc = (s0 + t0) + (s1_ + t1)
                    a, b_ = plsc.unpack(acc, format=plsc.PackFormat.INTERLEAVED)
                    outv[w, pl.ds(32 * g, 16)] = a
                    outv[w, pl.ds(32 * g + 16, 16)] = b_
                return c2

            lax.fori_loop(0, chunk, word_body, 0)
            out_copy(k, b).start()

        def pair_body(ii, carry):
            for u in (0, 1):
                i = 2 * ii + u

                @pl.when(i < n_chunks)
                def _():
                    s1(i, u)

                @pl.when(jnp.logical_and(i >= 1, i <= n_chunks))
                def _():
                    s2(i - 1, 1 - u)

                @pl.when(jnp.logical_and(i >= 2, i <= n_chunks + 1))
                def _():
                    s3(i - 2, u)
            return carry

        lax.fori_loop(0, (n_chunks + 2) // 2, pair_body, 0)
        out_copy(n_chunks - 2, 0).wait()
        out_copy(n_chunks - 1, 1).wait()

    return run(x_flat, codes, cb_packed)


def kernel(x, codes, codebook):
    bsz, seq = x.shape
    total = bsz * seq
    x_flat = x.reshape(total).astype(jnp.int32)
    cb_packed = _pack_codebook(codebook)
    out = _run(x_flat, codes, cb_packed, total)
    return out.reshape(bsz, seq, _D)
